# Initial kernel scaffold; baseline (speedup 1.0000x reference)
import jax
import jax.numpy as jnp
from jax.experimental import pallas as pl


def kernel(hidden_features, batch, current_epoch, edge_index, Ws1, bs1, g1, be1, Ws2, bs2, g2, be2, Ws3, bs3, Wf1, bf1, gf1, bef1, Wf2, bf2):
    raise NotImplementedError("write your pallas kernel here")


# speedup vs baseline: 1.0066x; 1.0066x over previous
import functools

import jax
import jax.numpy as jnp
from jax.experimental import pallas as pl

N_NODES = 10000
D_IN = 128
EMB_DIM = 16
R_PARAM = 0.3
GRAV_WEIGHT = 1.0

_BLK = 1000


def _ln(x, g, b, eps=1e-5):
    m = jnp.mean(x, axis=-1, keepdims=True)
    v = jnp.mean((x - m) ** 2, axis=-1, keepdims=True)
    return (x - m) * jax.lax.rsqrt(v + eps) * g + b


def _node_mlp_kernel(x_ref, ws1, bs1, g1, be1, ws2, bs2, g2, be2, ws3, bs3,
                     h1_ref, sp_ref):
    x = x_ref[...]
    h1 = jnp.concatenate([x, jnp.mean(x, axis=-1, keepdims=True)], axis=-1)
    y = jax.nn.relu(_ln(jnp.dot(h1, ws1[...], preferred_element_type=jnp.float32) + bs1[...], g1[...], be1[...]))
    y = jax.nn.relu(_ln(jnp.dot(y, ws2[...], preferred_element_type=jnp.float32) + bs2[...], g2[...], be2[...]))
    sp = jnp.dot(y, ws3[...], preferred_element_type=jnp.float32) + bs3[...]
    nrm = jnp.sqrt(jnp.sum(sp * sp, axis=-1, keepdims=True))
    sp = sp / jnp.maximum(nrm, 1e-12)
    h1_ref[...] = h1
    sp_ref[...] = sp


def _out_mlp_kernel(ssum_ref, cnt_ref, smax_ref, h1_ref, wf1, bf1, gf1, bef1,
                    wf2, bf2, out_ref):
    ssum = ssum_ref[...]
    cnt = cnt_ref[...]
    smean = ssum / jnp.maximum(cnt, 1.0)
    smax = jnp.where(cnt > 0, smax_ref[...], 0.0)
    cat = jnp.concatenate([ssum, smean, smax, h1_ref[...]], axis=-1)
    y = jax.nn.relu(_ln(jnp.dot(cat, wf1[...], preferred_element_type=jnp.float32) + bf1[...], gf1[...], bef1[...]))
    out_ref[...] = jnp.dot(y, wf2[...], preferred_element_type=jnp.float32) + bf2[...]


def _full_spec(shape):
    return pl.BlockSpec(shape, lambda i: tuple(0 for _ in shape))


def kernel(hidden_features, batch, current_epoch, edge_index, Ws1, bs1, g1, be1,
           Ws2, bs2, g2, be2, Ws3, bs3, Wf1, bf1, gf1, bef1, Wf2, bf2):
    n = hidden_features.shape[0]
    grid = (n // _BLK,)

    row_spec = lambda c: pl.BlockSpec((_BLK, c), lambda i: (i, 0))
    h1, sp = pl.pallas_call(
        _node_mlp_kernel,
        grid=grid,
        in_specs=[row_spec(D_IN)] + [_full_spec(w.shape) for w in
                                     (Ws1, bs1, g1, be1, Ws2, bs2, g2, be2, Ws3, bs3)],
        out_specs=[row_spec(D_IN + 1), row_spec(EMB_DIM)],
        out_shape=[jax.ShapeDtypeStruct((n, D_IN + 1), jnp.float32),
                   jax.ShapeDtypeStruct((n, EMB_DIM), jnp.float32)],
    )(hidden_features, Ws1, bs1, g1, be1, Ws2, bs2, g2, be2, Ws3, bs3)

    start = edge_index[0]
    end = edge_index[1]
    d = jnp.sum((sp[start] - sp[end]) ** 2, axis=-1)
    grav = jnp.exp(-GRAV_WEIGHT * d / (R_PARAM ** 2))
    he = h1[start] * grav[:, None]
    ssum = jax.ops.segment_sum(he, end, num_segments=n)
    cnt = jax.ops.segment_sum(jnp.ones_like(d), end, num_segments=n)
    smax = jax.ops.segment_max(he, end, num_segments=n)

    out = pl.pallas_call(
        _out_mlp_kernel,
        grid=grid,
        in_specs=[row_spec(D_IN + 1), row_spec(1), row_spec(D_IN + 1),
                  row_spec(D_IN + 1)] + [_full_spec(w.shape) for w in
                                         (Wf1, bf1, gf1, bef1, Wf2, bf2)],
        out_specs=row_spec(D_IN),
        out_shape=jax.ShapeDtypeStruct((n, D_IN), jnp.float32),
    )(ssum, cnt[:, None], smax, h1, Wf1, bf1, gf1, bef1, Wf2, bf2)

    return (out, edge_index, sp, jnp.float32(GRAV_WEIGHT))


# speedup vs baseline: 1.2532x; 1.2450x over previous
import functools

import jax
import jax.numpy as jnp
from jax import lax
from jax.experimental import pallas as pl
from jax.experimental.pallas import tpu as pltpu
from jax.experimental.pallas import tpu_sc as plsc

N_NODES = 10000
D_IN = 128
EMB_DIM = 16
R_PARAM = 0.3
GRAV_WEIGHT = 1.0

_BLK = 1000
_H1P = 144
_NSL = _H1P // 16
_CB = 128
_NW = 32
_NPAD = 10240
_OWN = _NPAD // _NW
_NEG = -3.0e38


def _ln(x, g, b, eps=1e-5):
    m = jnp.mean(x, axis=-1, keepdims=True)
    v = jnp.mean((x - m) ** 2, axis=-1, keepdims=True)
    return (x - m) * jax.lax.rsqrt(v + eps) * g + b


def _node_mlp_kernel(x_ref, ws1, bs1, g1, be1, ws2, bs2, g2, be2, ws3, bs3,
                     h1_ref, sp_ref):
    x = x_ref[...]
    h1 = jnp.concatenate(
        [x, jnp.mean(x, axis=-1, keepdims=True),
         jnp.zeros((x.shape[0], _H1P - D_IN - 1), jnp.float32)], axis=-1)
    y = jax.nn.relu(_ln(jnp.dot(h1[:, :D_IN + 1], ws1[...], preferred_element_type=jnp.float32) + bs1[...], g1[...], be1[...]))
    y = jax.nn.relu(_ln(jnp.dot(y, ws2[...], preferred_element_type=jnp.float32) + bs2[...], g2[...], be2[...]))
    sp = jnp.dot(y, ws3[...], preferred_element_type=jnp.float32) + bs3[...]
    nrm = jnp.sqrt(jnp.sum(sp * sp, axis=-1, keepdims=True))
    sp = sp / jnp.maximum(nrm, 1e-12)
    h1_ref[...] = h1
    sp_ref[...] = sp


def _out_mlp_kernel(ssum_ref, cnt_ref, smax_ref, h1_ref, wf1, bf1, gf1, bef1,
                    wf2, bf2, out_ref):
    ssum = ssum_ref[...]
    cnt = cnt_ref[...]
    smean = ssum / jnp.maximum(cnt, 1.0)
    smax = jnp.where(cnt > 0, smax_ref[...], 0.0)
    cat = jnp.concatenate([ssum, smean, smax, h1_ref[...]], axis=-1)
    y = jax.nn.relu(_ln(jnp.dot(cat, wf1[...], preferred_element_type=jnp.float32) + bf1[...], gf1[...], bef1[...]))
    out_ref[...] = jnp.dot(y, wf2[...], preferred_element_type=jnp.float32) + bf2[...]


def _full_spec(shape):
    return pl.BlockSpec(shape, lambda i: tuple(0 for _ in shape))


def _sc_edge_kernel(h1_hbm, sp_hbm, st_hbm, en_hbm, bd_hbm,
                    sum_hbm, max_hbm, cnt_hbm,
                    bounds_v, sidx_v, eidx_v, rows_v, sps_v, spe_v,
                    acc_s, acc_m, acc_c, sem1, sem2, sem3):
    info = plsc.get_sparse_core_info()
    nc = info.num_cores
    wid = lax.axis_index("s") * nc + lax.axis_index("c")
    dst_lo = wid * _OWN
    scale = jnp.float32(-GRAV_WEIGHT / (R_PARAM * R_PARAM))

    def init_body(r, carry):
        for j in range(_NSL):
            acc_s[r, pl.ds(j * 16, 16)] = jnp.zeros((16,), jnp.float32)
            acc_m[r, pl.ds(j * 16, 16)] = jnp.full((16,), _NEG, jnp.float32)
        acc_c[r, :] = jnp.zeros((16,), jnp.float32)
        return carry

    lax.fori_loop(0, _OWN, init_body, 0)

    pltpu.sync_copy(bd_hbm, bounds_v)
    e_lo = bounds_v[pl.ds(wid, 16)][0]
    e_hi = bounds_v[pl.ds(wid + 1, 16)][0]
    lo8 = (e_lo // 8) * 8
    nch = (e_hi - lo8 + (_CB - 1)) // _CB

    def edge_body16(i16, carry):
        base = i16 * 16
        rows16 = eidx_v[pl.ds(base, 16)] - dst_lo
        for k in range(16):
            row = rows16[k]

            @pl.when(jnp.logical_and(row >= 0, row < _OWN))
            def _(k=k, row=row):
                e = base + k
                df = sps_v[e, :] - spe_v[e, :]
                d = jnp.sum(df * df)
                gv = jnp.exp(jnp.full((16,), d * scale, jnp.float32))
                for j in range(_NSL):
                    sl = pl.ds(j * 16, 16)
                    c = gv * rows_v[e, sl]
                    acc_s[row, sl] = acc_s[row, sl] + c
                    acc_m[row, sl] = jnp.maximum(acc_m[row, sl], c)
                acc_c[row, :] = acc_c[row, :] + 1.0

        return carry

    def chunk_body(b, carry):
        ofs = pl.multiple_of(lo8 + b * _CB, 8)
        pltpu.sync_copy(st_hbm.at[pl.ds(ofs, _CB)], sidx_v)
        pltpu.sync_copy(en_hbm.at[pl.ds(ofs, _CB)], eidx_v)
        g1 = pltpu.async_copy(h1_hbm.at[sidx_v], rows_v, sem1)
        g2 = pltpu.async_copy(sp_hbm.at[sidx_v], sps_v, sem2)
        g3 = pltpu.async_copy(sp_hbm.at[eidx_v], spe_v, sem3)
        g1.wait()
        g2.wait()
        g3.wait()
        lax.fori_loop(0, _CB // 16, edge_body16, 0)
        return carry

    lax.fori_loop(0, nch, chunk_body, 0)

    pltpu.sync_copy(acc_s, sum_hbm.at[pl.ds(dst_lo, _OWN)])
    pltpu.sync_copy(acc_m, max_hbm.at[pl.ds(dst_lo, _OWN)])
    pltpu.sync_copy(acc_c, cnt_hbm.at[pl.ds(dst_lo, _OWN)])


def _sc_aggregate(h1pad, sp, starts, ends, bounds):
    mesh = plsc.VectorSubcoreMesh(core_axis_name="c", subcore_axis_name="s")
    kern = functools.partial(
        pl.kernel,
        mesh=mesh,
        compiler_params=pltpu.CompilerParams(
            needs_layout_passes=False, use_tc_tiling_on_sc=False),
        out_type=[
            jax.ShapeDtypeStruct((_NPAD, _H1P), jnp.float32),
            jax.ShapeDtypeStruct((_NPAD, _H1P), jnp.float32),
            jax.ShapeDtypeStruct((_NPAD, 16), jnp.float32),
        ],
        scratch_types=[
            pltpu.VMEM((48,), jnp.int32),
            pltpu.VMEM((_CB,), jnp.int32),
            pltpu.VMEM((_CB,), jnp.int32),
            pltpu.VMEM((_CB, _H1P), jnp.float32),
            pltpu.VMEM((_CB, EMB_DIM), jnp.float32),
            pltpu.VMEM((_CB, EMB_DIM), jnp.float32),
            pltpu.VMEM((_OWN, _H1P), jnp.float32),
            pltpu.VMEM((_OWN, _H1P), jnp.float32),
            pltpu.VMEM((_OWN, 16), jnp.float32),
            pltpu.SemaphoreType.DMA,
            pltpu.SemaphoreType.DMA,
            pltpu.SemaphoreType.DMA,
        ],
    )(_sc_edge_kernel)
    return kern(h1pad, sp, starts, ends, bounds)


def kernel(hidden_features, batch, current_epoch, edge_index, Ws1, bs1, g1, be1,
           Ws2, bs2, g2, be2, Ws3, bs3, Wf1, bf1, gf1, bef1, Wf2, bf2):
    n = hidden_features.shape[0]
    grid = (n // _BLK,)

    row_spec = lambda c: pl.BlockSpec((_BLK, c), lambda i: (i, 0))
    h1pad, sp = pl.pallas_call(
        _node_mlp_kernel,
        grid=grid,
        in_specs=[row_spec(D_IN)] + [_full_spec(w.shape) for w in
                                     (Ws1, bs1, g1, be1, Ws2, bs2, g2, be2, Ws3, bs3)],
        out_specs=[row_spec(_H1P), row_spec(EMB_DIM)],
        out_shape=[jax.ShapeDtypeStruct((n, _H1P), jnp.float32),
                   jax.ShapeDtypeStruct((n, EMB_DIM), jnp.float32)],
    )(hidden_features, Ws1, bs1, g1, be1, Ws2, bs2, g2, be2, Ws3, bs3)

    e = edge_index.shape[1]
    order = jnp.argsort(edge_index[1])
    starts = jnp.take(edge_index[0], order)
    ends = jnp.take(edge_index[1], order)
    e_pad = ((e + _CB - 1) // _CB) * _CB + _CB
    starts = jnp.concatenate(
        [starts, jnp.zeros((e_pad - e,), jnp.int32)])
    ends = jnp.concatenate(
        [ends, jnp.full((e_pad - e,), _NPAD - 1, jnp.int32)])
    bounds = jnp.searchsorted(
        ends, jnp.arange(_NW + 1, dtype=jnp.int32) * _OWN).astype(jnp.int32)
    bounds = jnp.concatenate(
        [bounds, jnp.zeros((48 - _NW - 1,), jnp.int32)])

    ssum_p, smax_p, cnt_p = _sc_aggregate(h1pad, sp, starts, ends, bounds)
    ssum = ssum_p[:n, :D_IN + 1]
    smax = smax_p[:n, :D_IN + 1]
    cnt = cnt_p[:n, 0]
    h1 = h1pad[:, :D_IN + 1]

    out = pl.pallas_call(
        _out_mlp_kernel,
        grid=grid,
        in_specs=[row_spec(D_IN + 1), row_spec(1), row_spec(D_IN + 1),
                  row_spec(D_IN + 1)] + [_full_spec(w.shape) for w in
                                         (Wf1, bf1, gf1, bef1, Wf2, bf2)],
        out_specs=row_spec(D_IN),
        out_shape=jax.ShapeDtypeStruct((n, D_IN), jnp.float32),
    )(ssum, cnt[:, None], smax, h1, Wf1, bf1, gf1, bef1, Wf2, bf2)

    return (out, edge_index, sp, jnp.float32(GRAV_WEIGHT))


# speedup vs baseline: 3.4818x; 2.7782x over previous
import functools

import jax
import jax.numpy as jnp
from jax import lax
from jax.experimental import pallas as pl
from jax.experimental.pallas import tpu as pltpu
from jax.experimental.pallas import tpu_sc as plsc

N_NODES = 10000
D_IN = 128
EMB_DIM = 16
R_PARAM = 0.3
GRAV_WEIGHT = 1.0

_BLK = 1000
_H1P = 144
_NSL = _H1P // 16
_CB = 128
_NW = 32
_NPAD = 10240
_OWN = _NPAD // _NW
_NEG = -3.0e38


def _ln(x, g, b, eps=1e-5):
    m = jnp.mean(x, axis=-1, keepdims=True)
    v = jnp.mean((x - m) ** 2, axis=-1, keepdims=True)
    return (x - m) * jax.lax.rsqrt(v + eps) * g + b


def _node_mlp_kernel(x_ref, ws1, bs1, g1, be1, ws2, bs2, g2, be2, ws3, bs3,
                     h1_ref, sp_ref):
    x = x_ref[...]
    h1 = jnp.concatenate(
        [x, jnp.mean(x, axis=-1, keepdims=True),
         jnp.zeros((x.shape[0], _H1P - D_IN - 1), jnp.float32)], axis=-1)
    y = jax.nn.relu(_ln(jnp.dot(h1[:, :D_IN + 1], ws1[...], preferred_element_type=jnp.float32) + bs1[...], g1[...], be1[...]))
    y = jax.nn.relu(_ln(jnp.dot(y, ws2[...], preferred_element_type=jnp.float32) + bs2[...], g2[...], be2[...]))
    sp = jnp.dot(y, ws3[...], preferred_element_type=jnp.float32) + bs3[...]
    nrm = jnp.sqrt(jnp.sum(sp * sp, axis=-1, keepdims=True))
    sp = sp / jnp.maximum(nrm, 1e-12)
    h1_ref[...] = h1
    sp_ref[...] = sp


def _out_mlp_kernel(ssum_ref, cnt_ref, smax_ref, h1_ref, wf1, bf1, gf1, bef1,
                    wf2, bf2, out_ref):
    ssum = ssum_ref[...]
    cnt = cnt_ref[...]
    smean = ssum / jnp.maximum(cnt, 1.0)
    smax = jnp.where(cnt > 0, smax_ref[...], 0.0)
    cat = jnp.concatenate([ssum, smean, smax, h1_ref[...]], axis=-1)
    y = jax.nn.relu(_ln(jnp.dot(cat, wf1[...], preferred_element_type=jnp.float32) + bf1[...], gf1[...], bef1[...]))
    out_ref[...] = jnp.dot(y, wf2[...], preferred_element_type=jnp.float32) + bf2[...]


def _full_spec(shape):
    return pl.BlockSpec(shape, lambda i: tuple(0 for _ in shape))


def _sc_edge_kernel(h1_hbm, sp_hbm, st_hbm, en_hbm, bd_hbm,
                    sum_hbm, max_hbm, cnt_hbm,
                    bounds_v, sidx_v, eidx_v, rows_v, sps_v, spe_v,
                    acc_s, acc_m, acc_c, sem1, sem2, sem3):
    info = plsc.get_sparse_core_info()
    nc = info.num_cores
    wid = lax.axis_index("s") * nc + lax.axis_index("c")
    dst_lo = wid * _OWN
    scale = jnp.float32(-GRAV_WEIGHT / (R_PARAM * R_PARAM))

    def init_body(r, carry):
        for j in range(_NSL):
            acc_s[r, pl.ds(j * 16, 16)] = jnp.zeros((16,), jnp.float32)
            acc_m[r, pl.ds(j * 16, 16)] = jnp.full((16,), _NEG, jnp.float32)
        acc_c[r, :] = jnp.zeros((16,), jnp.float32)
        return carry

    lax.fori_loop(0, _OWN, init_body, 0)

    pltpu.sync_copy(bd_hbm, bounds_v)
    e_lo = bounds_v[pl.ds(wid, 16)][0]
    e_hi = bounds_v[pl.ds(wid + 1, 16)][0]
    lo8 = (e_lo // 8) * 8
    nch = (e_hi - lo8 + (_CB - 1)) // _CB

    def edge_body16(i16, carry):
        base = i16 * 16
        rows16 = eidx_v[pl.ds(base, 16)] - dst_lo
        for k in range(16):
            row = rows16[k]

            @pl.when(jnp.logical_and(row >= 0, row < _OWN))
            def _(k=k, row=row):
                e = base + k
                df = sps_v[e, :] - spe_v[e, :]
                d = jnp.sum(df * df)
                gv = jnp.exp(jnp.full((16,), d * scale, jnp.float32))
                for j in range(_NSL):
                    sl = pl.ds(j * 16, 16)
                    c = gv * rows_v[e, sl]
                    acc_s[row, sl] = acc_s[row, sl] + c
                    acc_m[row, sl] = jnp.maximum(acc_m[row, sl], c)
                acc_c[row, :] = acc_c[row, :] + 1.0

        return carry

    def chunk_body(b, carry):
        ofs = pl.multiple_of(lo8 + b * _CB, 8)
        pltpu.sync_copy(st_hbm.at[pl.ds(ofs, _CB)], sidx_v)
        pltpu.sync_copy(en_hbm.at[pl.ds(ofs, _CB)], eidx_v)
        g1 = pltpu.async_copy(h1_hbm.at[sidx_v], rows_v, sem1)
        g2 = pltpu.async_copy(sp_hbm.at[sidx_v], sps_v, sem2)
        g3 = pltpu.async_copy(sp_hbm.at[eidx_v], spe_v, sem3)
        g1.wait()
        g2.wait()
        g3.wait()
        lax.fori_loop(0, _CB // 16, edge_body16, 0)
        return carry

    lax.fori_loop(0, nch, chunk_body, 0)

    pltpu.sync_copy(acc_s, sum_hbm.at[pl.ds(dst_lo, _OWN)])
    pltpu.sync_copy(acc_m, max_hbm.at[pl.ds(dst_lo, _OWN)])
    pltpu.sync_copy(acc_c, cnt_hbm.at[pl.ds(dst_lo, _OWN)])


def _sc_aggregate(h1pad, sp, starts, ends, bounds):
    mesh = plsc.VectorSubcoreMesh(core_axis_name="c", subcore_axis_name="s")
    kern = functools.partial(
        pl.kernel,
        mesh=mesh,
        compiler_params=pltpu.CompilerParams(
            needs_layout_passes=False, use_tc_tiling_on_sc=False),
        out_type=[
            jax.ShapeDtypeStruct((_NPAD, _H1P), jnp.float32),
            jax.ShapeDtypeStruct((_NPAD, _H1P), jnp.float32),
            jax.ShapeDtypeStruct((_NPAD, 16), jnp.float32),
        ],
        scratch_types=[
            pltpu.VMEM((48,), jnp.int32),
            pltpu.VMEM((_CB,), jnp.int32),
            pltpu.VMEM((_CB,), jnp.int32),
            pltpu.VMEM((_CB, _H1P), jnp.float32),
            pltpu.VMEM((_CB, EMB_DIM), jnp.float32),
            pltpu.VMEM((_CB, EMB_DIM), jnp.float32),
            pltpu.VMEM((_OWN, _H1P), jnp.float32),
            pltpu.VMEM((_OWN, _H1P), jnp.float32),
            pltpu.VMEM((_OWN, 16), jnp.float32),
            pltpu.SemaphoreType.DMA,
            pltpu.SemaphoreType.DMA,
            pltpu.SemaphoreType.DMA,
        ],
    )(_sc_edge_kernel)
    return kern(h1pad, sp, starts, ends, bounds)


def kernel(hidden_features, batch, current_epoch, edge_index, Ws1, bs1, g1, be1,
           Ws2, bs2, g2, be2, Ws3, bs3, Wf1, bf1, gf1, bef1, Wf2, bf2):
    n = hidden_features.shape[0]
    grid = (n // _BLK,)

    row_spec = lambda c: pl.BlockSpec((_BLK, c), lambda i: (i, 0))
    h1pad, sp = pl.pallas_call(
        _node_mlp_kernel,
        grid=grid,
        in_specs=[row_spec(D_IN)] + [_full_spec(w.shape) for w in
                                     (Ws1, bs1, g1, be1, Ws2, bs2, g2, be2, Ws3, bs3)],
        out_specs=[row_spec(_H1P), row_spec(EMB_DIM)],
        out_shape=[jax.ShapeDtypeStruct((n, _H1P), jnp.float32),
                   jax.ShapeDtypeStruct((n, EMB_DIM), jnp.float32)],
    )(hidden_features, Ws1, bs1, g1, be1, Ws2, bs2, g2, be2, Ws3, bs3)

    e = edge_index.shape[1]
    ends, starts = lax.sort((edge_index[1], edge_index[0]), num_keys=1)
    e_pad = ((e + _CB - 1) // _CB) * _CB + _CB
    starts = jnp.concatenate(
        [starts, jnp.zeros((e_pad - e,), jnp.int32)])
    ends = jnp.concatenate(
        [ends, jnp.full((e_pad - e,), _NPAD - 1, jnp.int32)])
    bounds = jnp.searchsorted(
        ends, jnp.arange(_NW + 1, dtype=jnp.int32) * _OWN).astype(jnp.int32)
    bounds = jnp.concatenate(
        [bounds, jnp.zeros((48 - _NW - 1,), jnp.int32)])

    ssum_p, smax_p, cnt_p = _sc_aggregate(h1pad, sp, starts, ends, bounds)
    ssum = ssum_p[:n, :D_IN + 1]
    smax = smax_p[:n, :D_IN + 1]
    cnt = cnt_p[:n, 0]
    h1 = h1pad[:, :D_IN + 1]

    out = pl.pallas_call(
        _out_mlp_kernel,
        grid=grid,
        in_specs=[row_spec(D_IN + 1), row_spec(1), row_spec(D_IN + 1),
                  row_spec(D_IN + 1)] + [_full_spec(w.shape) for w in
                                         (Wf1, bf1, gf1, bef1, Wf2, bf2)],
        out_specs=row_spec(D_IN),
        out_shape=jax.ShapeDtypeStruct((n, D_IN), jnp.float32),
    )(ssum, cnt[:, None], smax, h1, Wf1, bf1, gf1, bef1, Wf2, bf2)

    return (out, edge_index, sp, jnp.float32(GRAV_WEIGHT))
